# baseline (speedup 1.0000x reference)
import math

import jax
import jax.numpy as jnp
from jax import lax
from jax.experimental import pallas as pl
from jax.experimental.pallas import tpu as pltpu
from jax.experimental.pallas import tpu_sc as plsc

F32 = jnp.float32

N_NODES = 10000
N_EDGES = 320000
N_GRAPHS = 256
H = 128
GF = 128

_MEANS = [-1.645, -1.080, -0.739, -0.468, -0.228, 0.0, 0.228, 0.468, 0.739, 1.080, 1.645]
_STDS = [0.283, 0.170, 0.134, 0.118, 0.114, 0.114, 0.114, 0.118, 0.134, 0.170, 0.283]

_NC = 2
_NS = 16
_NW = _NC * _NS
_GC = 80
_EPW = N_EDGES // _NW
_EPT = N_EDGES // _NS
_RPT = 624
_REM_OFF = _NS * _RPT
_REM = N_NODES - _REM_OFF

def _sc_mesh():
    return plsc.VectorSubcoreMesh(
        core_axis_name="c", subcore_axis_name="s", num_cores=_NC, num_subcores=_NS
    )


def _sc_gather(lr, srcv, dstv, gc):
    ne = srcv.shape[0]
    epw = ne // _NW
    nch = epw // gc

    def gather_body(lr, srci, dsti, outs, outd,
                    idxs, idxd, rs0, rd0, rs1, rd1,
                    gss0, gss1, gsd0, gsd1, oss0, oss1, osd0, osd1):
        wid = lax.axis_index("s") * _NC + lax.axis_index("c")
        base = wid * epw
        rs = (rs0, rs1)
        rd = (rd0, rd1)
        gsem = (gss0, gss1)
        gdem = (gsd0, gsd1)
        osem = (oss0, oss1)
        odem = (osd0, osd1)

        pltpu.sync_copy(srci.at[pl.ds(base, epw)], idxs)
        pltpu.sync_copy(dsti.at[pl.ds(base, epw)], idxd)

        def start_gather(j, b):
            pltpu.async_copy(lr.at[idxs.at[pl.ds(j * gc, gc)]], rs[b], gsem[b])
            pltpu.async_copy(lr.at[idxd.at[pl.ds(j * gc, gc)]], rd[b], gdem[b])

        def wait_gather(j, b):
            pltpu.make_async_copy(lr.at[idxs.at[pl.ds(j * gc, gc)]], rs[b], gsem[b]).wait()
            pltpu.make_async_copy(lr.at[idxd.at[pl.ds(j * gc, gc)]], rd[b], gdem[b]).wait()

        def start_out(j, b):
            off = base + j * gc
            pltpu.async_copy(rs[b], outs.at[pl.ds(off, gc)], osem[b])
            pltpu.async_copy(rd[b], outd.at[pl.ds(off, gc)], odem[b])

        def wait_out(j, b):
            off = base + j * gc
            pltpu.make_async_copy(rs[b], outs.at[pl.ds(off, gc)], osem[b]).wait()
            pltpu.make_async_copy(rd[b], outd.at[pl.ds(off, gc)], odem[b]).wait()

        start_gather(0, 0)

        def body(k, carry):
            for b in (0, 1):
                j = 2 * k + b
                nb = 1 - b
                wait_gather(j, b)

                @pl.when(j >= 1)
                def _():
                    wait_out(j - 1, nb)

                @pl.when(j + 1 < nch)
                def _():
                    start_gather(j + 1, nb)

                start_out(j, b)
            return carry

        lax.fori_loop(0, nch // 2, body, 0)
        if nch % 2:
            jlast = nch - 1
            wait_gather(jlast, 0)
            wait_out(jlast - 1, 1)
            start_out(jlast, 0)
            wait_out(jlast, 0)
        else:
            wait_out(nch - 1, 1)

    f = pl.kernel(
        gather_body,
        out_type=(
            jax.ShapeDtypeStruct((ne, 2 * H), F32),
            jax.ShapeDtypeStruct((ne, 2 * H), F32),
        ),
        mesh=_sc_mesh(),
        scratch_types=(
            [pltpu.VMEM((epw,), jnp.int32)] * 2
            + [pltpu.VMEM((gc, 2 * H), F32)] * 4
            + [pltpu.SemaphoreType.DMA] * 8
        ),
    )
    return f(lr, srcv, dstv)


def _sc_scatter(v1, v2, zz, dstv):
    ne = dstv.shape[0]
    ept = ne // _NS
    nch = ept // _GC

    def scatter_body(v1, v2, zz, dsti, out,
                     iv0, iv1, iv2, iv3, vv0, vv1, vv2, vv3,
                     isem0, isem1, isem2, isem3,
                     rsem0, rsem1, rsem2, rsem3,
                     ssem0, ssem1, ssem2, ssem3, acc):
        c = lax.axis_index("c")
        s = lax.axis_index("s")
        iv = (iv0, iv1, iv2, iv3)
        vv = (vv0, vv1, vv2, vv3)
        isem = (isem0, isem1, isem2, isem3)
        rsem = (rsem0, rsem1, rsem2, rsem3)
        ssem = (ssem0, ssem1, ssem2, ssem3)

        pltpu.sync_copy(zz.at[pl.ds(s * _RPT, _RPT)], acc.at[pl.ds(s * _RPT, _RPT)])

        @pl.when(s == 0)
        def _():
            pltpu.sync_copy(zz.at[pl.ds(_REM_OFF, _REM)], acc.at[pl.ds(_REM_OFF, _REM)])

        plsc.subcore_barrier()

        def start_read(j, b):
            off = s * ept + j * _GC
            pltpu.async_copy(dsti.at[pl.ds(off, _GC)], iv[b], isem[b])

            @pl.when(c == 0)
            def _():
                pltpu.async_copy(v1.at[pl.ds(off, _GC)], vv[b], rsem[b])

            @pl.when(c != 0)
            def _():
                pltpu.async_copy(v2.at[pl.ds(off, _GC)], vv[b], rsem[b])

        def wait_read(j, b):
            off = s * ept + j * _GC
            pltpu.make_async_copy(dsti.at[pl.ds(off, _GC)], iv[b], isem[b]).wait()
            pltpu.make_async_copy(v1.at[pl.ds(off, _GC)], vv[b], rsem[b]).wait()

        def start_scat(j, b):
            pltpu.async_copy(vv[b], acc.at[iv[b]], ssem[b], add=True)

        def wait_scat(j, b):
            pltpu.make_async_copy(vv[b], acc.at[iv[b]], ssem[b]).wait()

        start_read(0, 0)
        start_read(1, 1)

        def body(k, carry):
            for b in (0, 1, 2, 3):
                j = 4 * k + b
                nxt = (b + 2) % 4
                wait_read(j, b)

                @pl.when(j >= 2)
                def _():
                    wait_scat(j - 2, nxt)

                @pl.when(j + 2 < nch)
                def _():
                    start_read(j + 2, nxt)

                start_scat(j, b)
            return carry

        lax.fori_loop(0, nch // 4, body, 0)
        for j in range((nch // 4) * 4, nch):
            b = j % 4
            wait_read(j, b)
            wait_scat(j - 2, (j - 2) % 4)
            start_scat(j, b)
        wait_scat(nch - 2, (nch - 2) % 4)
        wait_scat(nch - 1, (nch - 1) % 4)
        plsc.subcore_barrier()
        pltpu.sync_copy(
            acc.at[pl.ds(s * _RPT, _RPT)],
            out.at[pl.ds(c * N_NODES + s * _RPT, _RPT)],
        )

        @pl.when(s == 0)
        def _():
            pltpu.sync_copy(
                acc.at[pl.ds(_REM_OFF, _REM)],
                out.at[pl.ds(c * N_NODES + _REM_OFF, _REM)],
            )

    f = pl.kernel(
        scatter_body,
        out_type=jax.ShapeDtypeStruct((2 * N_NODES, H), F32),
        mesh=_sc_mesh(),
        scratch_types=(
            [pltpu.VMEM((_GC,), jnp.int32)] * 4
            + [pltpu.VMEM((_GC, H), F32)] * 4
            + [pltpu.SemaphoreType.DMA] * 12
            + [pltpu.VMEM_SHARED((N_NODES, H), F32)]
        ),
    )
    return f(v1, v2, zz, dstv)


def _rne16(x):
    return jnp.right_shift(x + 0x7FFF + (jnp.right_shift(x, 16) & 1), 16)


def _node_proj_body(nf, w, b, out1, out2):
    y = jnp.dot(nf[...], w[...], preferred_element_type=F32) + b[...]
    out1[...] = jnp.maximum(y[:, :H], 0.0)
    out2[...] = y[:, H:]


def _node_proj(nf, wcat, bcat):
    nb = 1000
    return pl.pallas_call(
        _node_proj_body,
        grid=(N_NODES // nb,),
        in_specs=[
            pl.BlockSpec((nb, H), lambda i: (i, 0)),
            pl.BlockSpec((H, 3 * H), lambda i: (0, 0)),
            pl.BlockSpec((1, 3 * H), lambda i: (0, 0)),
        ],
        out_specs=(
            pl.BlockSpec((nb, H), lambda i: (i, 0)),
            pl.BlockSpec((nb, 2 * H), lambda i: (i, 0)),
        ),
        out_shape=(
            jax.ShapeDtypeStruct((N_NODES, H), F32),
            jax.ShapeDtypeStruct((N_NODES, 2 * H), F32),
        ),
    )(nf, wcat, bcat)


def _relu(x):
    return jnp.maximum(x, 0.0)


_BF = jnp.bfloat16


def _split(a):
    ah = a.astype(_BF)
    al = (a - ah.astype(F32)).astype(_BF)
    return ah, al


def _dot3(a, bh, bl):
    ah, al = _split(a)
    return (
        jnp.dot(ah, bh, preferred_element_type=F32)
        + jnp.dot(ah, bl, preferred_element_type=F32)
        + jnp.dot(al, bh, preferred_element_type=F32)
    )


def _edge_body(gs, gd, ef, wee, bee, wn2eh, wn2el, bn2e, wueh, wuel, bue,
               w2e2nh, w2e2nl, b2e2n, out1, out2):
    y = jnp.dot(ef[...], wee[...], preferred_element_type=F32) + bee[...]
    e2n1 = _relu(y[:, :H])
    ee = _relu(y[:, H:])
    first = _relu(gs[:, :H] + gd[:, H:])
    second = _relu(gs[:, H:] + gd[:, :H])
    ne = _relu(
        _dot3(first, wn2eh[:H], wn2el[:H])
        + _dot3(second, wn2eh[H:], wn2el[H:])
        + bn2e[...]
    )
    nef = _relu(
        _dot3(ne, wueh[:H], wuel[:H])
        + _dot3(ee, wueh[H:], wuel[H:])
        + bue[...]
    )
    out1[...] = e2n1
    out2[...] = _relu(_dot3(nef, w2e2nh[...], w2e2nl[...]) + b2e2n[...])


def _edge_pipe(gs, gd, ef, wee, bee, wn2e, bn2e, wue, bue, w2e2n, b2e2n):
    ne = gs.shape[0]
    eb = 2560
    wspec = lambda r, c: pl.BlockSpec((r, c), lambda i: (0, 0))
    wn2eh, wn2el = _split(wn2e)
    wueh, wuel = _split(wue)
    w2e2nh, w2e2nl = _split(w2e2n)
    return pl.pallas_call(
        _edge_body,
        grid=(ne // eb,),
        in_specs=[
            pl.BlockSpec((eb, 2 * H), lambda i: (i, 0)),
            pl.BlockSpec((eb, 2 * H), lambda i: (i, 0)),
            pl.BlockSpec((eb, 16), lambda i: (i, 0)),
            wspec(16, 2 * H),
            wspec(1, 2 * H),
            wspec(2 * H, H),
            wspec(2 * H, H),
            wspec(1, H),
            wspec(2 * H, H),
            wspec(2 * H, H),
            wspec(1, H),
            wspec(H, H),
            wspec(H, H),
            wspec(1, H),
        ],
        out_specs=(
            pl.BlockSpec((eb, H), lambda i: (i, 0)),
            pl.BlockSpec((eb, H), lambda i: (i, 0)),
        ),
        out_shape=(
            jax.ShapeDtypeStruct((ne, H), F32),
            jax.ShapeDtypeStruct((ne, H), F32),
        ),
    )(gs, gd, ef, wee, bee, wn2eh, wn2el, bn2e, wueh, wuel, bue,
      w2e2nh, w2e2nl, b2e2n)


def _node2_body(nn1, en1a, en1b, en2a, en2b, w1un, b1un, w2n2n, b2n2n, w2un, b2un, wn2g, bn2g, h_out, sums):
    i = pl.program_id(0)
    en1 = en1a[...] + en1b[...]
    en2 = en2a[...] + en2b[...]
    z1 = _relu(
        jnp.dot(nn1[...], w1un[:H], preferred_element_type=F32)
        + jnp.dot(en1, w1un[H:], preferred_element_type=F32)
        + b1un[...]
    )
    nn2 = _relu(jnp.dot(z1, w2n2n[...], preferred_element_type=F32) + b2n2n[...])
    z2 = _relu(
        jnp.dot(nn2, w2un[:H], preferred_element_type=F32)
        + jnp.dot(en2, w2un[H:], preferred_element_type=F32)
        + b2un[...]
    )
    h = jnp.tanh(jnp.dot(z2, wn2g[...], preferred_element_type=F32) + bn2g[...])
    h_out[...] = h

    @pl.when(i == 0)
    def _():
        sums[...] = jnp.zeros_like(sums)

    sums[0:1, :] += jnp.sum(h, axis=0, keepdims=True)
    sums[1:2, :] += jnp.sum(h * h, axis=0, keepdims=True)


def _node2(nn1, en1a, en1b, en2a, en2b, w1un, b1un, w2n2n, b2n2n, w2un, b2un, wn2g, bn2g):
    nb = 1000
    wspec = lambda r, c: pl.BlockSpec((r, c), lambda i: (0, 0))
    return pl.pallas_call(
        _node2_body,
        grid=(N_NODES // nb,),
        in_specs=[
            pl.BlockSpec((nb, H), lambda i: (i, 0)),
            pl.BlockSpec((nb, H), lambda i: (i, 0)),
            pl.BlockSpec((nb, H), lambda i: (i, 0)),
            pl.BlockSpec((nb, H), lambda i: (i, 0)),
            pl.BlockSpec((nb, H), lambda i: (i, 0)),
            wspec(2 * H, H),
            wspec(1, H),
            wspec(H, H),
            wspec(1, H),
            wspec(2 * H, H),
            wspec(1, H),
            wspec(H, GF),
            wspec(1, GF),
        ],
        out_specs=(
            pl.BlockSpec((nb, GF), lambda i: (i, 0)),
            pl.BlockSpec((8, GF), lambda i: (0, 0)),
        ),
        out_shape=(
            jax.ShapeDtypeStruct((N_NODES, GF), F32),
            jax.ShapeDtypeStruct((8, GF), F32),
        ),
    )(nn1, en1a, en1b, en2a, en2b, w1un, b1un, w2n2n, b2n2n, w2un, b2un, wn2g, bn2g)


def _readout_body(h, sums, gids, gamma, beta, wh, bout, wpred, bpred, out, gacc):
    i = pl.program_id(0)
    n_inv = 1.0 / N_NODES
    mean = sums[0:1, :] * n_inv
    var = sums[1:2, :] * n_inv - mean * mean
    inv = lax.rsqrt(var + 1e-5)
    hb = (h[...] - mean) * inv * gamma[...] + beta[...]

    ms = []
    denom = None
    for k in range(11):
        d = (hb - _MEANS[k]) * (1.0 / _STDS[k])
        mk = jnp.exp(-0.5 * d * d) * (1.0 / (_STDS[k] * math.sqrt(2.0 * math.pi)))
        ms.append(mk)
        denom = mk if denom is None else denom + mk
    rden = 1.0 / denom
    y = None
    for k in range(11):
        t = jnp.dot(ms[k] * rden, wh[k], preferred_element_type=F32)
        y = t if y is None else y + t

    ohT = (lax.broadcasted_iota(jnp.int32, (N_GRAPHS, h.shape[0]), 0) == gids[0]).astype(F32)

    @pl.when(i == 0)
    def _():
        gacc[...] = jnp.zeros_like(gacc)

    gacc[...] += jnp.dot(ohT, y, preferred_element_type=F32)

    @pl.when(i == pl.num_programs(0) - 1)
    def _():
        g = jnp.tanh(gacc[...] + bout[...])
        out[...] = jnp.dot(g, wpred[...], preferred_element_type=F32) + bpred[...]


def _readout(h, sums, gids3, gamma, beta, wh, bout, wpred, bpred):
    nb = 1000
    wspec = lambda r, c: pl.BlockSpec((r, c), lambda i: (0, 0))
    return pl.pallas_call(
        _readout_body,
        grid=(N_NODES // nb,),
        in_specs=[
            pl.BlockSpec((nb, GF), lambda i: (i, 0)),
            wspec(8, GF),
            pl.BlockSpec((1, 1, nb), lambda i: (i, 0, 0)),
            wspec(1, GF),
            wspec(1, GF),
            pl.BlockSpec((11, GF, GF), lambda i: (0, 0, 0)),
            wspec(1, GF),
            wspec(GF, 1),
            wspec(1, 1),
        ],
        out_specs=pl.BlockSpec((N_GRAPHS, 1), lambda i: (0, 0)),
        out_shape=jax.ShapeDtypeStruct((N_GRAPHS, 1), F32),
        scratch_shapes=[pltpu.VMEM((N_GRAPHS, GF), F32)],
    )(h, sums, gids3, gamma, beta, wh, bout, wpred, bpred)


def kernel(node_feats, edge_feats, params, edge_index, node_graph_ids):
    p = params
    src = edge_index[0]
    dst = edge_index[1]

    wnlr = jnp.concatenate([p["W_l1_n2n"], p["W_l1_l"], p["W_l1_r"]], axis=1)
    bnlr = jnp.concatenate([p["b_l1_n2n"], p["b_l1_l"], p["b_l1_r"]])[None]
    nn1, lr = _node_proj(node_feats, wnlr, bnlr)

    wee = jnp.concatenate([p["W_l1_e2n"], p["W_l1_e2e"]], axis=1)
    bee = jnp.concatenate([p["b_l1_e2n"], p["b_l1_e2e"]])[None]
    zz = jnp.zeros((N_NODES, H), F32)

    e2 = 62 * 32 * _GC
    accs = []
    for lo, hi in ((0, e2), (e2, N_EDGES)):
        gs, gd = _sc_gather(lr, src[lo:hi], dst[lo:hi], _GC)
        e2n1, e2n2 = _edge_pipe(
            gs, gd, edge_feats[lo:hi],
            wee, bee,
            p["W_l1_n2e"], p["b_l1_n2e"][None],
            p["W_l1_ue"], p["b_l1_ue"][None],
            p["W_l2_e2n"], p["b_l2_e2n"][None],
        )
        accs.append(_sc_scatter(e2n1, e2n2, zz, dst[lo:hi]))

    h, sums = _node2(
        nn1, accs[0][:N_NODES], accs[1][:N_NODES],
        accs[0][N_NODES:], accs[1][N_NODES:],
        p["W_l1_un"], p["b_l1_un"][None],
        p["W_l2_n2n"], p["b_l2_n2n"][None],
        p["W_l2_un"], p["b_l2_un"][None],
        p["W_n2g"], p["b_n2g"][None],
    )

    wh = p["W_out"].reshape(GF, 11, GF).transpose(1, 0, 2)
    gids3 = node_graph_ids.reshape(N_NODES // 1000, 1, 1000)
    out = _readout(
        h, sums, gids3,
        p["bn_gamma"][None], p["bn_beta"][None],
        wh, p["b_out"][None],
        p["W_pred"], p["b_pred"][None],
    )
    return out


# speedup vs baseline: 1.1012x; 1.1012x over previous
import math

import jax
import jax.numpy as jnp
from jax import lax
from jax.experimental import pallas as pl
from jax.experimental.pallas import tpu as pltpu
from jax.experimental.pallas import tpu_sc as plsc

F32 = jnp.float32

N_NODES = 10000
N_EDGES = 320000
N_GRAPHS = 256
H = 128
GF = 128

_MEANS = [-1.645, -1.080, -0.739, -0.468, -0.228, 0.0, 0.228, 0.468, 0.739, 1.080, 1.645]
_STDS = [0.283, 0.170, 0.134, 0.118, 0.114, 0.114, 0.114, 0.118, 0.134, 0.170, 0.283]

_NC = 2
_NS = 16
_NW = _NC * _NS
_GC = 80
_EPW = N_EDGES // _NW
_EPT = N_EDGES // _NS
_RPT = 624
_REM_OFF = _NS * _RPT
_REM = N_NODES - _REM_OFF

def _sc_mesh():
    return plsc.VectorSubcoreMesh(
        core_axis_name="c", subcore_axis_name="s", num_cores=_NC, num_subcores=_NS
    )


def _sc_gather(lr, srcv, dstv, gc):
    ne = srcv.shape[0]
    epw = ne // _NW
    nch = epw // gc

    def gather_body(lr, srci, dsti, outs, outd,
                    idxs, idxd, rs0, rd0, rs1, rd1,
                    gss0, gss1, gsd0, gsd1, oss0, oss1, osd0, osd1):
        wid = lax.axis_index("s") * _NC + lax.axis_index("c")
        base = wid * epw
        rs = (rs0, rs1)
        rd = (rd0, rd1)
        gsem = (gss0, gss1)
        gdem = (gsd0, gsd1)
        osem = (oss0, oss1)
        odem = (osd0, osd1)

        pltpu.sync_copy(srci.at[pl.ds(base, epw)], idxs)
        pltpu.sync_copy(dsti.at[pl.ds(base, epw)], idxd)

        def start_gather(j, b):
            pltpu.async_copy(lr.at[idxs.at[pl.ds(j * gc, gc)]], rs[b], gsem[b])
            pltpu.async_copy(lr.at[idxd.at[pl.ds(j * gc, gc)]], rd[b], gdem[b])

        def wait_gather(j, b):
            pltpu.make_async_copy(lr.at[idxs.at[pl.ds(j * gc, gc)]], rs[b], gsem[b]).wait()
            pltpu.make_async_copy(lr.at[idxd.at[pl.ds(j * gc, gc)]], rd[b], gdem[b]).wait()

        def start_out(j, b):
            off = base + j * gc
            pltpu.async_copy(rs[b], outs.at[pl.ds(off, gc)], osem[b])
            pltpu.async_copy(rd[b], outd.at[pl.ds(off, gc)], odem[b])

        def wait_out(j, b):
            off = base + j * gc
            pltpu.make_async_copy(rs[b], outs.at[pl.ds(off, gc)], osem[b]).wait()
            pltpu.make_async_copy(rd[b], outd.at[pl.ds(off, gc)], odem[b]).wait()

        start_gather(0, 0)

        def body(k, carry):
            for b in (0, 1):
                j = 2 * k + b
                nb = 1 - b
                wait_gather(j, b)

                @pl.when(j >= 1)
                def _():
                    wait_out(j - 1, nb)

                @pl.when(j + 1 < nch)
                def _():
                    start_gather(j + 1, nb)

                start_out(j, b)
            return carry

        lax.fori_loop(0, nch // 2, body, 0)
        if nch % 2:
            jlast = nch - 1
            wait_gather(jlast, 0)
            wait_out(jlast - 1, 1)
            start_out(jlast, 0)
            wait_out(jlast, 0)
        else:
            wait_out(nch - 1, 1)

    f = pl.kernel(
        gather_body,
        out_type=(
            jax.ShapeDtypeStruct((ne, 2 * H), F32),
            jax.ShapeDtypeStruct((ne, 2 * H), F32),
        ),
        mesh=_sc_mesh(),
        scratch_types=(
            [pltpu.VMEM((epw,), jnp.int32)] * 2
            + [pltpu.VMEM((gc, 2 * H), F32)] * 4
            + [pltpu.SemaphoreType.DMA] * 8
        ),
    )
    return f(lr, srcv, dstv)


def _sc_scatter(v1, v2, zz, dstv):
    ne = dstv.shape[0]
    ept = ne // _NS
    nch = ept // _GC

    def scatter_body(v1, v2, zz, dsti, out,
                     iv0, iv1, iv2, iv3, vv0, vv1, vv2, vv3,
                     isem0, isem1, isem2, isem3,
                     rsem0, rsem1, rsem2, rsem3,
                     ssem0, ssem1, ssem2, ssem3, acc):
        c = lax.axis_index("c")
        s = lax.axis_index("s")
        iv = (iv0, iv1, iv2, iv3)
        vv = (vv0, vv1, vv2, vv3)
        isem = (isem0, isem1, isem2, isem3)
        rsem = (rsem0, rsem1, rsem2, rsem3)
        ssem = (ssem0, ssem1, ssem2, ssem3)

        pltpu.sync_copy(zz.at[pl.ds(s * _RPT, _RPT)], acc.at[pl.ds(s * _RPT, _RPT)])

        @pl.when(s == 0)
        def _():
            pltpu.sync_copy(zz.at[pl.ds(_REM_OFF, _REM)], acc.at[pl.ds(_REM_OFF, _REM)])

        plsc.subcore_barrier()

        def start_read(j, b):
            off = s * ept + j * _GC
            pltpu.async_copy(dsti.at[pl.ds(off, _GC)], iv[b], isem[b])

            @pl.when(c == 0)
            def _():
                pltpu.async_copy(v1.at[pl.ds(off, _GC)], vv[b], rsem[b])

            @pl.when(c != 0)
            def _():
                pltpu.async_copy(v2.at[pl.ds(off, _GC)], vv[b], rsem[b])

        def wait_read(j, b):
            off = s * ept + j * _GC
            pltpu.make_async_copy(dsti.at[pl.ds(off, _GC)], iv[b], isem[b]).wait()
            pltpu.make_async_copy(v1.at[pl.ds(off, _GC)], vv[b], rsem[b]).wait()

        def start_scat(j, b):
            pltpu.async_copy(vv[b], acc.at[iv[b]], ssem[b], add=True)

        def wait_scat(j, b):
            pltpu.make_async_copy(vv[b], acc.at[iv[b]], ssem[b]).wait()

        start_read(0, 0)
        start_read(1, 1)

        def body(k, carry):
            for b in (0, 1, 2, 3):
                j = 4 * k + b
                nxt = (b + 2) % 4
                wait_read(j, b)

                @pl.when(j >= 2)
                def _():
                    wait_scat(j - 2, nxt)

                @pl.when(j + 2 < nch)
                def _():
                    start_read(j + 2, nxt)

                start_scat(j, b)
            return carry

        lax.fori_loop(0, nch // 4, body, 0)
        for j in range((nch // 4) * 4, nch):
            b = j % 4
            wait_read(j, b)
            wait_scat(j - 2, (j - 2) % 4)
            start_scat(j, b)
        wait_scat(nch - 2, (nch - 2) % 4)
        wait_scat(nch - 1, (nch - 1) % 4)
        plsc.subcore_barrier()
        pltpu.sync_copy(
            acc.at[pl.ds(s * _RPT, _RPT)],
            out.at[pl.ds(c * N_NODES + s * _RPT, _RPT)],
        )

        @pl.when(s == 0)
        def _():
            pltpu.sync_copy(
                acc.at[pl.ds(_REM_OFF, _REM)],
                out.at[pl.ds(c * N_NODES + _REM_OFF, _REM)],
            )

    f = pl.kernel(
        scatter_body,
        out_type=jax.ShapeDtypeStruct((2 * N_NODES, H), F32),
        mesh=_sc_mesh(),
        scratch_types=(
            [pltpu.VMEM((_GC,), jnp.int32)] * 4
            + [pltpu.VMEM((_GC, H), F32)] * 4
            + [pltpu.SemaphoreType.DMA] * 12
            + [pltpu.VMEM_SHARED((N_NODES, H), F32)]
        ),
    )
    return f(v1, v2, zz, dstv)


def _rne16(x):
    return jnp.right_shift(x + 0x7FFF + (jnp.right_shift(x, 16) & 1), 16)


def _node_proj_body(nf, w, b, out1, out2):
    y = jnp.dot(nf[...], w[...], preferred_element_type=F32) + b[...]
    out1[...] = jnp.maximum(y[:, :H], 0.0)
    out2[...] = y[:, H:]


def _node_proj(nf, wcat, bcat):
    nb = 1000
    return pl.pallas_call(
        _node_proj_body,
        grid=(N_NODES // nb,),
        in_specs=[
            pl.BlockSpec((nb, H), lambda i: (i, 0)),
            pl.BlockSpec((H, 3 * H), lambda i: (0, 0)),
            pl.BlockSpec((1, 3 * H), lambda i: (0, 0)),
        ],
        out_specs=(
            pl.BlockSpec((nb, H), lambda i: (i, 0)),
            pl.BlockSpec((nb, 2 * H), lambda i: (i, 0)),
        ),
        out_shape=(
            jax.ShapeDtypeStruct((N_NODES, H), F32),
            jax.ShapeDtypeStruct((N_NODES, 2 * H), F32),
        ),
    )(nf, wcat, bcat)


def _relu(x):
    return jnp.maximum(x, 0.0)


_BF = jnp.bfloat16


def _split(a):
    ah = a.astype(_BF)
    al = (a - ah.astype(F32)).astype(_BF)
    return ah, al


def _dot3(a, bh, bl):
    ah, al = _split(a)
    return (
        jnp.dot(ah, bh, preferred_element_type=F32)
        + jnp.dot(ah, bl, preferred_element_type=F32)
        + jnp.dot(al, bh, preferred_element_type=F32)
    )


def _edge_body(gs, gd, ef, wee, bee, wn2e, bn2e, wue, bue, w2e2n, b2e2n, out1, out2):
    y = jnp.dot(ef[...], wee[...], preferred_element_type=F32) + bee[...]
    e2n1 = _relu(y[:, :H])
    ee = _relu(y[:, H:])
    first = _relu(gs[:, :H] + gd[:, H:])
    second = _relu(gs[:, H:] + gd[:, :H])
    ne = _relu(
        jnp.dot(first, wn2e[:H], preferred_element_type=F32)
        + jnp.dot(second, wn2e[H:], preferred_element_type=F32)
        + bn2e[...]
    )
    nef = _relu(
        jnp.dot(ne, wue[:H], preferred_element_type=F32)
        + jnp.dot(ee, wue[H:], preferred_element_type=F32)
        + bue[...]
    )
    out1[...] = e2n1
    out2[...] = _relu(jnp.dot(nef, w2e2n[...], preferred_element_type=F32) + b2e2n[...])


def _edge_pipe(gs, gd, ef, wee, bee, wn2e, bn2e, wue, bue, w2e2n, b2e2n):
    ne = gs.shape[0]
    eb = 2560
    wspec = lambda r, c: pl.BlockSpec((r, c), lambda i: (0, 0))
    return pl.pallas_call(
        _edge_body,
        grid=(ne // eb,),
        in_specs=[
            pl.BlockSpec((eb, 2 * H), lambda i: (i, 0)),
            pl.BlockSpec((eb, 2 * H), lambda i: (i, 0)),
            pl.BlockSpec((eb, 16), lambda i: (i, 0)),
            wspec(16, 2 * H),
            wspec(1, 2 * H),
            wspec(2 * H, H),
            wspec(1, H),
            wspec(2 * H, H),
            wspec(1, H),
            wspec(H, H),
            wspec(1, H),
        ],
        out_specs=(
            pl.BlockSpec((eb, H), lambda i: (i, 0)),
            pl.BlockSpec((eb, H), lambda i: (i, 0)),
        ),
        out_shape=(
            jax.ShapeDtypeStruct((ne, H), F32),
            jax.ShapeDtypeStruct((ne, H), F32),
        ),
    )(gs, gd, ef, wee, bee, wn2e, bn2e, wue, bue, w2e2n, b2e2n)


def _node2_body(nn1, en1a, en1b, en2a, en2b, w1un, b1un, w2n2n, b2n2n, w2un, b2un, wn2g, bn2g, h_out, sums):
    i = pl.program_id(0)
    en1 = en1a[...] + en1b[...]
    en2 = en2a[...] + en2b[...]
    z1 = _relu(
        jnp.dot(nn1[...], w1un[:H], preferred_element_type=F32)
        + jnp.dot(en1, w1un[H:], preferred_element_type=F32)
        + b1un[...]
    )
    nn2 = _relu(jnp.dot(z1, w2n2n[...], preferred_element_type=F32) + b2n2n[...])
    z2 = _relu(
        jnp.dot(nn2, w2un[:H], preferred_element_type=F32)
        + jnp.dot(en2, w2un[H:], preferred_element_type=F32)
        + b2un[...]
    )
    h = jnp.tanh(jnp.dot(z2, wn2g[...], preferred_element_type=F32) + bn2g[...])
    h_out[...] = h

    @pl.when(i == 0)
    def _():
        sums[...] = jnp.zeros_like(sums)

    sums[0:1, :] += jnp.sum(h, axis=0, keepdims=True)
    sums[1:2, :] += jnp.sum(h * h, axis=0, keepdims=True)


def _node2(nn1, en1a, en1b, en2a, en2b, w1un, b1un, w2n2n, b2n2n, w2un, b2un, wn2g, bn2g):
    nb = 1000
    wspec = lambda r, c: pl.BlockSpec((r, c), lambda i: (0, 0))
    return pl.pallas_call(
        _node2_body,
        grid=(N_NODES // nb,),
        in_specs=[
            pl.BlockSpec((nb, H), lambda i: (i, 0)),
            pl.BlockSpec((nb, H), lambda i: (i, 0)),
            pl.BlockSpec((nb, H), lambda i: (i, 0)),
            pl.BlockSpec((nb, H), lambda i: (i, 0)),
            pl.BlockSpec((nb, H), lambda i: (i, 0)),
            wspec(2 * H, H),
            wspec(1, H),
            wspec(H, H),
            wspec(1, H),
            wspec(2 * H, H),
            wspec(1, H),
            wspec(H, GF),
            wspec(1, GF),
        ],
        out_specs=(
            pl.BlockSpec((nb, GF), lambda i: (i, 0)),
            pl.BlockSpec((8, GF), lambda i: (0, 0)),
        ),
        out_shape=(
            jax.ShapeDtypeStruct((N_NODES, GF), F32),
            jax.ShapeDtypeStruct((8, GF), F32),
        ),
    )(nn1, en1a, en1b, en2a, en2b, w1un, b1un, w2n2n, b2n2n, w2un, b2un, wn2g, bn2g)


def _readout_body(h, sums, gids, gamma, beta, wh, bout, wpred, bpred, out, gacc):
    i = pl.program_id(0)
    n_inv = 1.0 / N_NODES
    mean = sums[0:1, :] * n_inv
    var = sums[1:2, :] * n_inv - mean * mean
    inv = lax.rsqrt(var + 1e-5)
    hb = (h[...] - mean) * inv * gamma[...] + beta[...]

    ms = []
    denom = None
    for k in range(11):
        d = (hb - _MEANS[k]) * (1.0 / _STDS[k])
        mk = jnp.exp(-0.5 * d * d) * (1.0 / (_STDS[k] * math.sqrt(2.0 * math.pi)))
        ms.append(mk)
        denom = mk if denom is None else denom + mk
    rden = 1.0 / denom
    y = None
    for k in range(11):
        t = jnp.dot(ms[k] * rden, wh[k], preferred_element_type=F32)
        y = t if y is None else y + t

    ohT = (lax.broadcasted_iota(jnp.int32, (N_GRAPHS, h.shape[0]), 0) == gids[0]).astype(F32)

    @pl.when(i == 0)
    def _():
        gacc[...] = jnp.zeros_like(gacc)

    gacc[...] += jnp.dot(ohT, y, preferred_element_type=F32)

    @pl.when(i == pl.num_programs(0) - 1)
    def _():
        g = jnp.tanh(gacc[...] + bout[...])
        out[...] = jnp.dot(g, wpred[...], preferred_element_type=F32) + bpred[...]


def _readout(h, sums, gids3, gamma, beta, wh, bout, wpred, bpred):
    nb = 1000
    wspec = lambda r, c: pl.BlockSpec((r, c), lambda i: (0, 0))
    return pl.pallas_call(
        _readout_body,
        grid=(N_NODES // nb,),
        in_specs=[
            pl.BlockSpec((nb, GF), lambda i: (i, 0)),
            wspec(8, GF),
            pl.BlockSpec((1, 1, nb), lambda i: (i, 0, 0)),
            wspec(1, GF),
            wspec(1, GF),
            pl.BlockSpec((11, GF, GF), lambda i: (0, 0, 0)),
            wspec(1, GF),
            wspec(GF, 1),
            wspec(1, 1),
        ],
        out_specs=pl.BlockSpec((N_GRAPHS, 1), lambda i: (0, 0)),
        out_shape=jax.ShapeDtypeStruct((N_GRAPHS, 1), F32),
        scratch_shapes=[pltpu.VMEM((N_GRAPHS, GF), F32)],
    )(h, sums, gids3, gamma, beta, wh, bout, wpred, bpred)


def kernel(node_feats, edge_feats, params, edge_index, node_graph_ids):
    p = params
    src = edge_index[0]
    dst = edge_index[1]

    wnlr = jnp.concatenate([p["W_l1_n2n"], p["W_l1_l"], p["W_l1_r"]], axis=1)
    bnlr = jnp.concatenate([p["b_l1_n2n"], p["b_l1_l"], p["b_l1_r"]])[None]
    nn1, lr = _node_proj(node_feats, wnlr, bnlr)

    wee = jnp.concatenate([p["W_l1_e2n"], p["W_l1_e2e"]], axis=1)
    bee = jnp.concatenate([p["b_l1_e2n"], p["b_l1_e2e"]])[None]
    zz = jnp.zeros((N_NODES, H), F32)

    e2 = 62 * 32 * _GC
    accs = []
    for lo, hi in ((0, e2), (e2, N_EDGES)):
        gs, gd = _sc_gather(lr, src[lo:hi], dst[lo:hi], _GC)
        e2n1, e2n2 = _edge_pipe(
            gs, gd, edge_feats[lo:hi],
            wee, bee,
            p["W_l1_n2e"], p["b_l1_n2e"][None],
            p["W_l1_ue"], p["b_l1_ue"][None],
            p["W_l2_e2n"], p["b_l2_e2n"][None],
        )
        accs.append(_sc_scatter(e2n1, e2n2, zz, dst[lo:hi]))

    h, sums = _node2(
        nn1, accs[0][:N_NODES], accs[1][:N_NODES],
        accs[0][N_NODES:], accs[1][N_NODES:],
        p["W_l1_un"], p["b_l1_un"][None],
        p["W_l2_n2n"], p["b_l2_n2n"][None],
        p["W_l2_un"], p["b_l2_un"][None],
        p["W_n2g"], p["b_n2g"][None],
    )

    wh = p["W_out"].reshape(GF, 11, GF).transpose(1, 0, 2)
    gids3 = node_graph_ids.reshape(N_NODES // 1000, 1, 1000)
    out = _readout(
        h, sums, gids3,
        p["bn_gamma"][None], p["bn_beta"][None],
        wh, p["b_out"][None],
        p["W_pred"], p["b_pred"][None],
    )
    return out


# speedup vs baseline: 1.1221x; 1.0190x over previous
import math

import jax
import jax.numpy as jnp
from jax import lax
from jax.experimental import pallas as pl
from jax.experimental.pallas import tpu as pltpu
from jax.experimental.pallas import tpu_sc as plsc

F32 = jnp.float32

N_NODES = 10000
N_EDGES = 320000
N_GRAPHS = 256
H = 128
GF = 128

_MEANS = [-1.645, -1.080, -0.739, -0.468, -0.228, 0.0, 0.228, 0.468, 0.739, 1.080, 1.645]
_STDS = [0.283, 0.170, 0.134, 0.118, 0.114, 0.114, 0.114, 0.118, 0.134, 0.170, 0.283]

_NC = 2
_NS = 16
_NW = _NC * _NS
_GC = 80
_EPW = N_EDGES // _NW
_EPT = N_EDGES // _NS
_RPT = 624
_REM_OFF = _NS * _RPT
_REM = N_NODES - _REM_OFF

def _sc_mesh():
    return plsc.VectorSubcoreMesh(
        core_axis_name="c", subcore_axis_name="s", num_cores=_NC, num_subcores=_NS
    )


def _sc_gather(lr, srcv, dstv, gc):
    ne = srcv.shape[0]
    epw = ne // _NW
    nch = epw // gc

    def gather_body(lr, srci, dsti, outs, outd,
                    idxs, idxd, rs0, rd0, rs1, rd1,
                    gss0, gss1, gsd0, gsd1, oss0, oss1, osd0, osd1):
        wid = lax.axis_index("s") * _NC + lax.axis_index("c")
        base = wid * epw
        rs = (rs0, rs1)
        rd = (rd0, rd1)
        gsem = (gss0, gss1)
        gdem = (gsd0, gsd1)
        osem = (oss0, oss1)
        odem = (osd0, osd1)

        pltpu.sync_copy(srci.at[pl.ds(base, epw)], idxs)
        pltpu.sync_copy(dsti.at[pl.ds(base, epw)], idxd)

        def start_gather(j, b):
            pltpu.async_copy(lr.at[idxs.at[pl.ds(j * gc, gc)]], rs[b], gsem[b])
            pltpu.async_copy(lr.at[idxd.at[pl.ds(j * gc, gc)]], rd[b], gdem[b])

        def wait_gather(j, b):
            pltpu.make_async_copy(lr.at[idxs.at[pl.ds(j * gc, gc)]], rs[b], gsem[b]).wait()
            pltpu.make_async_copy(lr.at[idxd.at[pl.ds(j * gc, gc)]], rd[b], gdem[b]).wait()

        def start_out(j, b):
            off = base + j * gc
            pltpu.async_copy(rs[b], outs.at[pl.ds(off, gc)], osem[b])
            pltpu.async_copy(rd[b], outd.at[pl.ds(off, gc)], odem[b])

        def wait_out(j, b):
            off = base + j * gc
            pltpu.make_async_copy(rs[b], outs.at[pl.ds(off, gc)], osem[b]).wait()
            pltpu.make_async_copy(rd[b], outd.at[pl.ds(off, gc)], odem[b]).wait()

        start_gather(0, 0)

        def body(k, carry):
            for b in (0, 1):
                j = 2 * k + b
                nb = 1 - b
                wait_gather(j, b)

                @pl.when(j >= 1)
                def _():
                    wait_out(j - 1, nb)

                @pl.when(j + 1 < nch)
                def _():
                    start_gather(j + 1, nb)

                start_out(j, b)
            return carry

        lax.fori_loop(0, nch // 2, body, 0)
        if nch % 2:
            jlast = nch - 1
            wait_gather(jlast, 0)
            wait_out(jlast - 1, 1)
            start_out(jlast, 0)
            wait_out(jlast, 0)
        else:
            wait_out(nch - 1, 1)

    f = pl.kernel(
        gather_body,
        out_type=(
            jax.ShapeDtypeStruct((ne, 2 * H), F32),
            jax.ShapeDtypeStruct((ne, 2 * H), F32),
        ),
        mesh=_sc_mesh(),
        scratch_types=(
            [pltpu.VMEM((epw,), jnp.int32)] * 2
            + [pltpu.VMEM((gc, 2 * H), F32)] * 4
            + [pltpu.SemaphoreType.DMA] * 8
        ),
    )
    return f(lr, srcv, dstv)


def _sc_scatter(v1, v2, zz, dstv):
    ne = dstv.shape[0]
    ept = ne // _NS
    nch = ept // _GC

    def scatter_body(v1, v2, zz, dsti, out,
                     iv0, iv1, iv2, iv3, vv0, vv1, vv2, vv3,
                     isem0, isem1, isem2, isem3,
                     rsem0, rsem1, rsem2, rsem3,
                     ssem0, ssem1, ssem2, ssem3, acc):
        c = lax.axis_index("c")
        s = lax.axis_index("s")
        iv = (iv0, iv1, iv2, iv3)
        vv = (vv0, vv1, vv2, vv3)
        isem = (isem0, isem1, isem2, isem3)
        rsem = (rsem0, rsem1, rsem2, rsem3)
        ssem = (ssem0, ssem1, ssem2, ssem3)

        pltpu.sync_copy(
            zz.at[pl.ds(c * N_NODES + s * _RPT, _RPT)],
            acc.at[pl.ds(s * _RPT, _RPT)],
        )

        @pl.when(s == 0)
        def _():
            pltpu.sync_copy(
                zz.at[pl.ds(c * N_NODES + _REM_OFF, _REM)],
                acc.at[pl.ds(_REM_OFF, _REM)],
            )

        plsc.subcore_barrier()

        def start_read(j, b):
            off = s * ept + j * _GC
            pltpu.async_copy(dsti.at[pl.ds(off, _GC)], iv[b], isem[b])

            @pl.when(c == 0)
            def _():
                pltpu.async_copy(v1.at[pl.ds(off, _GC)], vv[b], rsem[b])

            @pl.when(c != 0)
            def _():
                pltpu.async_copy(v2.at[pl.ds(off, _GC)], vv[b], rsem[b])

        def wait_read(j, b):
            off = s * ept + j * _GC
            pltpu.make_async_copy(dsti.at[pl.ds(off, _GC)], iv[b], isem[b]).wait()
            pltpu.make_async_copy(v1.at[pl.ds(off, _GC)], vv[b], rsem[b]).wait()

        def start_scat(j, b):
            pltpu.async_copy(vv[b], acc.at[iv[b]], ssem[b], add=True)

        def wait_scat(j, b):
            pltpu.make_async_copy(vv[b], acc.at[iv[b]], ssem[b]).wait()

        start_read(0, 0)
        start_read(1, 1)

        def body(k, carry):
            for b in (0, 1, 2, 3):
                j = 4 * k + b
                nxt = (b + 2) % 4
                wait_read(j, b)

                @pl.when(j >= 2)
                def _():
                    wait_scat(j - 2, nxt)

                @pl.when(j + 2 < nch)
                def _():
                    start_read(j + 2, nxt)

                start_scat(j, b)
            return carry

        lax.fori_loop(0, nch // 4, body, 0)
        for j in range((nch // 4) * 4, nch):
            b = j % 4
            wait_read(j, b)
            wait_scat(j - 2, (j - 2) % 4)
            start_scat(j, b)
        wait_scat(nch - 2, (nch - 2) % 4)
        wait_scat(nch - 1, (nch - 1) % 4)
        plsc.subcore_barrier()
        pltpu.sync_copy(
            acc.at[pl.ds(s * _RPT, _RPT)],
            out.at[pl.ds(c * N_NODES + s * _RPT, _RPT)],
        )

        @pl.when(s == 0)
        def _():
            pltpu.sync_copy(
                acc.at[pl.ds(_REM_OFF, _REM)],
                out.at[pl.ds(c * N_NODES + _REM_OFF, _REM)],
            )

    f = pl.kernel(
        scatter_body,
        out_type=jax.ShapeDtypeStruct((2 * N_NODES, H), F32),
        mesh=_sc_mesh(),
        scratch_types=(
            [pltpu.VMEM((_GC,), jnp.int32)] * 4
            + [pltpu.VMEM((_GC, H), F32)] * 4
            + [pltpu.SemaphoreType.DMA] * 12
            + [pltpu.VMEM_SHARED((N_NODES, H), F32)]
        ),
    )
    return f(v1, v2, zz, dstv)


def _rne16(x):
    return jnp.right_shift(x + 0x7FFF + (jnp.right_shift(x, 16) & 1), 16)


def _node_proj_body(nf, w, b, out1, out2):
    y = jnp.dot(nf[...], w[...], preferred_element_type=F32) + b[...]
    out1[...] = jnp.maximum(y[:, :H], 0.0)
    out2[...] = y[:, H:]


def _node_proj(nf, wcat, bcat):
    nb = 1000
    return pl.pallas_call(
        _node_proj_body,
        grid=(N_NODES // nb,),
        in_specs=[
            pl.BlockSpec((nb, H), lambda i: (i, 0)),
            pl.BlockSpec((H, 3 * H), lambda i: (0, 0)),
            pl.BlockSpec((1, 3 * H), lambda i: (0, 0)),
        ],
        out_specs=(
            pl.BlockSpec((nb, H), lambda i: (i, 0)),
            pl.BlockSpec((nb, 2 * H), lambda i: (i, 0)),
        ),
        out_shape=(
            jax.ShapeDtypeStruct((N_NODES, H), F32),
            jax.ShapeDtypeStruct((N_NODES, 2 * H), F32),
        ),
    )(nf, wcat, bcat)


def _relu(x):
    return jnp.maximum(x, 0.0)


_BF = jnp.bfloat16


def _split(a):
    ah = a.astype(_BF)
    al = (a - ah.astype(F32)).astype(_BF)
    return ah, al


def _dot3(a, bh, bl):
    ah, al = _split(a)
    return (
        jnp.dot(ah, bh, preferred_element_type=F32)
        + jnp.dot(ah, bl, preferred_element_type=F32)
        + jnp.dot(al, bh, preferred_element_type=F32)
    )


def _edge_body(gs, gd, ef, wee, bee, wn2e, bn2e, wue, bue, w2e2n, b2e2n, out1, out2):
    y = jnp.dot(ef[...], wee[...], preferred_element_type=F32) + bee[...]
    e2n1 = _relu(y[:, :H])
    ee = _relu(y[:, H:])
    first = _relu(gs[:, :H] + gd[:, H:])
    second = _relu(gs[:, H:] + gd[:, :H])
    ne = _relu(
        jnp.dot(first, wn2e[:H], preferred_element_type=F32)
        + jnp.dot(second, wn2e[H:], preferred_element_type=F32)
        + bn2e[...]
    )
    nef = _relu(
        jnp.dot(ne, wue[:H], preferred_element_type=F32)
        + jnp.dot(ee, wue[H:], preferred_element_type=F32)
        + bue[...]
    )
    out1[...] = e2n1
    out2[...] = _relu(jnp.dot(nef, w2e2n[...], preferred_element_type=F32) + b2e2n[...])


def _edge_pipe(gs, gd, ef, wee, bee, wn2e, bn2e, wue, bue, w2e2n, b2e2n):
    ne = gs.shape[0]
    eb = 2560
    wspec = lambda r, c: pl.BlockSpec((r, c), lambda i: (0, 0))
    return pl.pallas_call(
        _edge_body,
        grid=(ne // eb,),
        in_specs=[
            pl.BlockSpec((eb, 2 * H), lambda i: (i, 0)),
            pl.BlockSpec((eb, 2 * H), lambda i: (i, 0)),
            pl.BlockSpec((eb, 16), lambda i: (i, 0)),
            wspec(16, 2 * H),
            wspec(1, 2 * H),
            wspec(2 * H, H),
            wspec(1, H),
            wspec(2 * H, H),
            wspec(1, H),
            wspec(H, H),
            wspec(1, H),
        ],
        out_specs=(
            pl.BlockSpec((eb, H), lambda i: (i, 0)),
            pl.BlockSpec((eb, H), lambda i: (i, 0)),
        ),
        out_shape=(
            jax.ShapeDtypeStruct((ne, H), F32),
            jax.ShapeDtypeStruct((ne, H), F32),
        ),
    )(gs, gd, ef, wee, bee, wn2e, bn2e, wue, bue, w2e2n, b2e2n)


def _node2_body(nn1, en1, en2, w1un, b1un, w2n2n, b2n2n, w2un, b2un, wn2g, bn2g, h_out, sums):
    i = pl.program_id(0)
    z1 = _relu(
        jnp.dot(nn1[...], w1un[:H], preferred_element_type=F32)
        + jnp.dot(en1[...], w1un[H:], preferred_element_type=F32)
        + b1un[...]
    )
    nn2 = _relu(jnp.dot(z1, w2n2n[...], preferred_element_type=F32) + b2n2n[...])
    z2 = _relu(
        jnp.dot(nn2, w2un[:H], preferred_element_type=F32)
        + jnp.dot(en2[...], w2un[H:], preferred_element_type=F32)
        + b2un[...]
    )
    h = jnp.tanh(jnp.dot(z2, wn2g[...], preferred_element_type=F32) + bn2g[...])
    h_out[...] = h

    @pl.when(i == 0)
    def _():
        sums[...] = jnp.zeros_like(sums)

    sums[0:1, :] += jnp.sum(h, axis=0, keepdims=True)
    sums[1:2, :] += jnp.sum(h * h, axis=0, keepdims=True)


def _node2(nn1, en1, en2, w1un, b1un, w2n2n, b2n2n, w2un, b2un, wn2g, bn2g):
    nb = 1000
    wspec = lambda r, c: pl.BlockSpec((r, c), lambda i: (0, 0))
    return pl.pallas_call(
        _node2_body,
        grid=(N_NODES // nb,),
        in_specs=[
            pl.BlockSpec((nb, H), lambda i: (i, 0)),
            pl.BlockSpec((nb, H), lambda i: (i, 0)),
            pl.BlockSpec((nb, H), lambda i: (i, 0)),
            wspec(2 * H, H),
            wspec(1, H),
            wspec(H, H),
            wspec(1, H),
            wspec(2 * H, H),
            wspec(1, H),
            wspec(H, GF),
            wspec(1, GF),
        ],
        out_specs=(
            pl.BlockSpec((nb, GF), lambda i: (i, 0)),
            pl.BlockSpec((8, GF), lambda i: (0, 0)),
        ),
        out_shape=(
            jax.ShapeDtypeStruct((N_NODES, GF), F32),
            jax.ShapeDtypeStruct((8, GF), F32),
        ),
    )(nn1, en1, en2, w1un, b1un, w2n2n, b2n2n, w2un, b2un, wn2g, bn2g)


def _readout_body(h, sums, gids, gamma, beta, wh, bout, wpred, bpred, out, gacc):
    i = pl.program_id(0)
    n_inv = 1.0 / N_NODES
    mean = sums[0:1, :] * n_inv
    var = sums[1:2, :] * n_inv - mean * mean
    inv = lax.rsqrt(var + 1e-5)
    hb = (h[...] - mean) * inv * gamma[...] + beta[...]

    ms = []
    denom = None
    for k in range(11):
        d = (hb - _MEANS[k]) * (1.0 / _STDS[k])
        mk = jnp.exp(-0.5 * d * d) * (1.0 / (_STDS[k] * math.sqrt(2.0 * math.pi)))
        ms.append(mk)
        denom = mk if denom is None else denom + mk
    rden = 1.0 / denom
    y = None
    for k in range(11):
        t = jnp.dot(ms[k] * rden, wh[k], preferred_element_type=F32)
        y = t if y is None else y + t

    ohT = (lax.broadcasted_iota(jnp.int32, (N_GRAPHS, h.shape[0]), 0) == gids[0]).astype(F32)

    @pl.when(i == 0)
    def _():
        gacc[...] = jnp.zeros_like(gacc)

    gacc[...] += jnp.dot(ohT, y, preferred_element_type=F32)

    @pl.when(i == pl.num_programs(0) - 1)
    def _():
        g = jnp.tanh(gacc[...] + bout[...])
        out[...] = jnp.dot(g, wpred[...], preferred_element_type=F32) + bpred[...]


def _readout(h, sums, gids3, gamma, beta, wh, bout, wpred, bpred):
    nb = 1000
    wspec = lambda r, c: pl.BlockSpec((r, c), lambda i: (0, 0))
    return pl.pallas_call(
        _readout_body,
        grid=(N_NODES // nb,),
        in_specs=[
            pl.BlockSpec((nb, GF), lambda i: (i, 0)),
            wspec(8, GF),
            pl.BlockSpec((1, 1, nb), lambda i: (i, 0, 0)),
            wspec(1, GF),
            wspec(1, GF),
            pl.BlockSpec((11, GF, GF), lambda i: (0, 0, 0)),
            wspec(1, GF),
            wspec(GF, 1),
            wspec(1, 1),
        ],
        out_specs=pl.BlockSpec((N_GRAPHS, 1), lambda i: (0, 0)),
        out_shape=jax.ShapeDtypeStruct((N_GRAPHS, 1), F32),
        scratch_shapes=[pltpu.VMEM((N_GRAPHS, GF), F32)],
    )(h, sums, gids3, gamma, beta, wh, bout, wpred, bpred)


def kernel(node_feats, edge_feats, params, edge_index, node_graph_ids):
    p = params
    src = edge_index[0]
    dst = edge_index[1]

    wnlr = jnp.concatenate([p["W_l1_n2n"], p["W_l1_l"], p["W_l1_r"]], axis=1)
    bnlr = jnp.concatenate([p["b_l1_n2n"], p["b_l1_l"], p["b_l1_r"]])[None]
    nn1, lr = _node_proj(node_feats, wnlr, bnlr)

    wee = jnp.concatenate([p["W_l1_e2n"], p["W_l1_e2e"]], axis=1)
    bee = jnp.concatenate([p["b_l1_e2n"], p["b_l1_e2e"]])[None]

    unit = _NW * _GC
    cuts = (0, 31 * unit, 62 * unit, 93 * unit, N_EDGES)
    acc = jnp.zeros((2 * N_NODES, H), F32)
    for lo, hi in zip(cuts[:-1], cuts[1:]):
        gs, gd = _sc_gather(lr, src[lo:hi], dst[lo:hi], _GC)
        e2n1, e2n2 = _edge_pipe(
            gs, gd, edge_feats[lo:hi],
            wee, bee,
            p["W_l1_n2e"], p["b_l1_n2e"][None],
            p["W_l1_ue"], p["b_l1_ue"][None],
            p["W_l2_e2n"], p["b_l2_e2n"][None],
        )
        acc = _sc_scatter(e2n1, e2n2, acc, dst[lo:hi])

    h, sums = _node2(
        nn1, acc[:N_NODES], acc[N_NODES:],
        p["W_l1_un"], p["b_l1_un"][None],
        p["W_l2_n2n"], p["b_l2_n2n"][None],
        p["W_l2_un"], p["b_l2_un"][None],
        p["W_n2g"], p["b_n2g"][None],
    )

    wh = p["W_out"].reshape(GF, 11, GF).transpose(1, 0, 2)
    gids3 = node_graph_ids.reshape(N_NODES // 1000, 1, 1000)
    out = _readout(
        h, sums, gids3,
        p["bn_gamma"][None], p["bn_beta"][None],
        wh, p["b_out"][None],
        p["W_pred"], p["b_pred"][None],
    )
    return out
